# baseline (speedup 1.0000x reference)
import functools

import jax
import jax.numpy as jnp
from jax import lax
from jax.experimental import pallas as pl
from jax.experimental.pallas import tpu as pltpu
from jax.experimental.pallas import tpu_sc as plsc


def _make_gather(NB, V, DP, NC, NS):
    NW = NC * NS
    BPW = NB // NW
    C = 40
    NBUF = 2
    NCH = BPW // C

    mesh = plsc.VectorSubcoreMesh(core_axis_name="c", subcore_axis_name="s")

    @functools.partial(
        pl.kernel,
        mesh=mesh,
        out_type=jax.ShapeDtypeStruct((NB, DP), jnp.float32),
        scratch_types=[
            pltpu.VMEM((BPW,), jnp.int32),
            [pltpu.VMEM((C, DP), jnp.float32)] * NBUF,
            [pltpu.SemaphoreType.DMA] * NBUF,
            [pltpu.SemaphoreType.DMA] * NBUF,
        ],
    )
    def gather_kernel(idx_hbm, table_hbm, out_hbm, idx_v, rows, gsems, osems):
        wid = lax.axis_index("s") * NC + lax.axis_index("c")
        base = wid * BPW
        pltpu.sync_copy(idx_hbm.at[pl.ds(base, BPW)], idx_v)

        def gather_desc(j, b):
            return pltpu.make_async_copy(
                table_hbm.at[idx_v.at[pl.ds(j * C, C)]], rows[b], gsems[b]
            )

        def out_desc(j, b):
            return pltpu.make_async_copy(
                rows[b], out_hbm.at[pl.ds(base + j * C, C)], osems[b]
            )

        for b in range(NBUF):
            gather_desc(b, b).start()

        def body(p, _):
            j0 = p * NBUF
            for b in range(NBUF):
                gather_desc(j0 + b, b).wait()
                out_desc(j0 + b, b).start()
            for b in range(NBUF):
                out_desc(j0 + b, b).wait()
                gather_desc(j0 + NBUF + b, b).start()
            return 0

        lax.fori_loop(0, NCH // NBUF - 1, body, 0)

        j0 = NCH - NBUF
        for b in range(NBUF):
            gather_desc(j0 + b, b).wait()
            out_desc(j0 + b, b).start()
        for b in range(NBUF):
            out_desc(j0 + b, b).wait()

    return gather_kernel


def kernel(indices, table):
    B, H = indices.shape
    V, D = table.shape
    DP = (D + 127) // 128 * 128
    flat_idx = indices.reshape(B * H).astype(jnp.int32)
    table_p = jnp.pad(table, ((0, 0), (0, DP - D)))
    info = plsc.get_sparse_core_info()
    padded = _make_gather(B * H, V, DP, info.num_cores, info.num_subcores)(
        flat_idx, table_p
    )
    return padded[:, :D].reshape(B, H, D)
